# baseline (speedup 1.0000x reference)
import functools

import jax
import jax.numpy as jnp
from jax import lax
from jax.experimental import pallas as pl
from jax.experimental.pallas import tpu as pltpu
from jax.experimental.pallas import tpu_sc as plsc

EPS = 1e-15
N_NODES = 10000
N_PAD = 10240
D_FEAT = 128
N_EDGES = 320000
E_TOTAL = 2 * N_EDGES

NUM_CORES = 2
NUM_SUBCORES = 16
NUM_WORKERS = NUM_CORES * NUM_SUBCORES
EDGES_PER_WORKER = E_TOTAL // NUM_WORKERS
CHUNK = 80
NUM_CHUNKS = EDGES_PER_WORKER // CHUNK
QDEPTH = 8

_MM_STEPS = 25
_MM_BM = N_NODES // _MM_STEPS
_G_ROWS = N_NODES * N_PAD // 2 // D_FEAT
_G_BLOCK = _G_ROWS // _MM_STEPS


def _mm_body(a_ref, b_ref, out_ref):
    res = lax.dot_general(a_ref[...], b_ref[...],
                          (((1,), (1,)), ((), ())),
                          preferred_element_type=jnp.float32)
    b = lax.bitcast_convert_type(res, jnp.int32)
    t = lax.shift_right_logical(
        b + jnp.int32(0x7FFF) + (lax.shift_right_logical(b, 16) & 1), 16)
    lo = t[:, :N_PAD // 2]
    hi = t[:, N_PAD // 2:]
    w = lo | (hi << 16)
    out_ref[...] = w.reshape(_G_BLOCK, D_FEAT)


_tc_matmul = pl.pallas_call(
    _mm_body,
    grid=(_MM_STEPS,),
    in_specs=[
        pl.BlockSpec((_MM_BM, D_FEAT), lambda i: (i, 0)),
        pl.BlockSpec((N_PAD, D_FEAT), lambda i: (0, 0)),
    ],
    out_specs=pl.BlockSpec((_G_BLOCK, D_FEAT), lambda i: (i, 0)),
    out_shape=jax.ShapeDtypeStruct((_G_ROWS, D_FEAT), jnp.int32),
)


def _sc_gather_body(g_hbm, fidx_hbm, out_hbm, fidxv, outb, sem):
    wid = lax.axis_index("s") * NUM_CORES + lax.axis_index("c")
    ebase = wid * EDGES_PER_WORKER

    pltpu.sync_copy(fidx_hbm.at[pl.ds(ebase, EDGES_PER_WORKER)], fidxv)

    def stream(c):
        return pltpu.make_async_copy(
            g_hbm.at[fidxv.at[pl.ds(c * CHUNK, CHUNK)]],
            outb.at[pl.ds(c * CHUNK, CHUNK)], sem)

    def fire(c, carry):
        stream(c).start()
        return carry

    def fire_drain(c, carry):
        stream(c).start()
        stream(c - QDEPTH).wait()
        return carry

    def drain(c, carry):
        stream(c).wait()
        return carry

    lax.fori_loop(0, QDEPTH, fire, 0)
    lax.fori_loop(QDEPTH, NUM_CHUNKS, fire_drain, 0)
    lax.fori_loop(NUM_CHUNKS - QDEPTH, NUM_CHUNKS, drain, 0)

    pltpu.sync_copy(outb, out_hbm.at[pl.ds(ebase, EDGES_PER_WORKER)])


_sc_gather = pl.kernel(
    _sc_gather_body,
    out_type=jax.ShapeDtypeStruct((E_TOTAL,), jnp.int32),
    mesh=plsc.VectorSubcoreMesh(
        core_axis_name="c", subcore_axis_name="s",
        num_cores=NUM_CORES, num_subcores=NUM_SUBCORES,
    ),
    scratch_types=[
        pltpu.VMEM((EDGES_PER_WORKER,), jnp.int32),
        pltpu.VMEM((EDGES_PER_WORKER,), jnp.int32),
        pltpu.SemaphoreType.DMA,
    ],
)


_L_ROWS = E_TOTAL // D_FEAT


def _tc_loss_body(w_ref, par_ref, out_ref):
    w = w_ref[...]
    par = par_ref[...]
    bits = jnp.where(par == 0, w << 16, w & jnp.int32(-65536))
    x = lax.bitcast_convert_type(bits, jnp.float32)
    p = x[:_L_ROWS // 2, :]
    n = x[_L_ROWS // 2:, :]
    pos_l = -jnp.log(jax.nn.sigmoid(p) + EPS)
    neg_l = -jnp.log(1.0 - jax.nn.sigmoid(n) + EPS)
    out_ref[0, 0] = (jnp.sum(pos_l) + jnp.sum(neg_l)) / N_EDGES


_tc_loss = pl.pallas_call(
    _tc_loss_body,
    out_specs=pl.BlockSpec(memory_space=pltpu.SMEM),
    out_shape=jax.ShapeDtypeStruct((1, 1), jnp.float32),
)


def kernel(z, pos_edge_index, neg_edge_index):
    zb = z.astype(jnp.bfloat16)
    zpad = jnp.concatenate(
        [zb, jnp.zeros((N_PAD - N_NODES, D_FEAT), jnp.bfloat16)])
    g = _tc_matmul(zb, zpad).reshape(N_NODES * N_PAD // 2)
    src = jnp.concatenate([pos_edge_index[0], neg_edge_index[0]]
                          ).astype(jnp.int32)
    dst = jnp.concatenate([pos_edge_index[1], neg_edge_index[1]]
                          ).astype(jnp.int32)
    half = N_PAD // 2
    parity = (dst >= half).astype(jnp.int32)
    widx = src * half + (dst - parity * half)
    words = _sc_gather(g, widx)
    loss = _tc_loss(words.reshape(_L_ROWS, D_FEAT),
                    parity.reshape(_L_ROWS, D_FEAT))
    return loss[0, 0]


# speedup vs baseline: 1.2752x; 1.2752x over previous
import functools

import jax
import jax.numpy as jnp
from jax import lax
from jax.experimental import pallas as pl
from jax.experimental.pallas import tpu as pltpu
from jax.experimental.pallas import tpu_sc as plsc

EPS = 1e-15
N_NODES = 10000
N_PAD = 10240
D_FEAT = 128
N_EDGES = 320000
E_TOTAL = 2 * N_EDGES

NUM_CORES = 2
NUM_SUBCORES = 16
NUM_WORKERS = NUM_CORES * NUM_SUBCORES
EDGES_PER_WORKER = E_TOTAL // NUM_WORKERS
CHUNK = 80
NUM_CHUNKS = EDGES_PER_WORKER // CHUNK
QDEPTH = 8

_MM_STEPS = 25
_MM_BM = N_NODES // _MM_STEPS
_G_ROWS = N_NODES * N_PAD // 2 // D_FEAT
_G_BLOCK = _G_ROWS // _MM_STEPS


def _mm_body(a_ref, b_ref, out_ref):
    res = lax.dot_general(a_ref[...], b_ref[...],
                          (((1,), (1,)), ((), ())),
                          preferred_element_type=jnp.float32)
    b = lax.bitcast_convert_type(res, jnp.int32)
    lo = lax.shift_right_logical(b[:, :N_PAD // 2], 16)
    hi = b[:, N_PAD // 2:] & jnp.int32(-65536)
    w = lo | hi
    out_ref[...] = w.reshape(_G_BLOCK, D_FEAT)


_tc_matmul = pl.pallas_call(
    _mm_body,
    grid=(_MM_STEPS,),
    in_specs=[
        pl.BlockSpec((_MM_BM, D_FEAT), lambda i: (i, 0)),
        pl.BlockSpec((N_PAD, D_FEAT), lambda i: (0, 0)),
    ],
    out_specs=pl.BlockSpec((_G_BLOCK, D_FEAT), lambda i: (i, 0)),
    out_shape=jax.ShapeDtypeStruct((_G_ROWS, D_FEAT), jnp.int32),
)


def _sc_gather_body(g_hbm, fidx_hbm, out_hbm, fidxv, outb, sem):
    wid = lax.axis_index("s") * NUM_CORES + lax.axis_index("c")
    ebase = wid * EDGES_PER_WORKER

    pltpu.sync_copy(fidx_hbm.at[pl.ds(ebase, EDGES_PER_WORKER)], fidxv)

    def stream(c):
        return pltpu.make_async_copy(
            g_hbm.at[fidxv.at[pl.ds(c * CHUNK, CHUNK)]],
            outb.at[pl.ds(c * CHUNK, CHUNK)], sem)

    def fire(c, carry):
        stream(c).start()
        return carry

    def fire_drain(c, carry):
        stream(c).start()
        stream(c - QDEPTH).wait()
        return carry

    def drain(c, carry):
        stream(c).wait()
        return carry

    lax.fori_loop(0, QDEPTH, fire, 0)
    lax.fori_loop(QDEPTH, NUM_CHUNKS, fire_drain, 0)
    lax.fori_loop(NUM_CHUNKS - QDEPTH, NUM_CHUNKS, drain, 0)

    pltpu.sync_copy(outb, out_hbm.at[pl.ds(ebase, EDGES_PER_WORKER)])


_sc_gather = pl.kernel(
    _sc_gather_body,
    out_type=jax.ShapeDtypeStruct((E_TOTAL,), jnp.int32),
    mesh=plsc.VectorSubcoreMesh(
        core_axis_name="c", subcore_axis_name="s",
        num_cores=NUM_CORES, num_subcores=NUM_SUBCORES,
    ),
    scratch_types=[
        pltpu.VMEM((EDGES_PER_WORKER,), jnp.int32),
        pltpu.VMEM((EDGES_PER_WORKER,), jnp.int32),
        pltpu.SemaphoreType.DMA,
    ],
)


_L_ROWS = E_TOTAL // D_FEAT


def _tc_loss_body(w_ref, par_ref, out_ref):
    w = w_ref[...]
    par = par_ref[...]
    bits = jnp.where(par == 0, w << 16, w & jnp.int32(-65536))
    x = lax.bitcast_convert_type(bits, jnp.float32)
    p = x[:_L_ROWS // 2, :]
    n = x[_L_ROWS // 2:, :]
    pos_l = -jnp.log(jax.nn.sigmoid(p) + EPS)
    neg_l = -jnp.log(1.0 - jax.nn.sigmoid(n) + EPS)
    out_ref[0, 0] = (jnp.sum(pos_l) + jnp.sum(neg_l)) / N_EDGES


_tc_loss = pl.pallas_call(
    _tc_loss_body,
    out_specs=pl.BlockSpec(memory_space=pltpu.SMEM),
    out_shape=jax.ShapeDtypeStruct((1, 1), jnp.float32),
)


def kernel(z, pos_edge_index, neg_edge_index):
    zb = z.astype(jnp.bfloat16)
    zpad = jnp.concatenate(
        [zb, jnp.zeros((N_PAD - N_NODES, D_FEAT), jnp.bfloat16)])
    g = _tc_matmul(zb, zpad).reshape(N_NODES * N_PAD // 2)
    src = jnp.concatenate([pos_edge_index[0], neg_edge_index[0]]
                          ).astype(jnp.int32)
    dst = jnp.concatenate([pos_edge_index[1], neg_edge_index[1]]
                          ).astype(jnp.int32)
    half = N_PAD // 2
    parity = (dst >= half).astype(jnp.int32)
    widx = src * half + (dst - parity * half)
    words = _sc_gather(g, widx)
    loss = _tc_loss(words.reshape(_L_ROWS, D_FEAT),
                    parity.reshape(_L_ROWS, D_FEAT))
    return loss[0, 0]
